# baseline (speedup 1.0000x reference)
import functools

import jax
import jax.numpy as jnp
from jax import lax
from jax.experimental import pallas as pl
from jax.experimental.pallas import tpu as pltpu, tpu_sc as plsc


def _sc_info():
    info = plsc.get_sparse_core_info()
    return info.num_cores, info.num_subcores


def _sc_gather(aug, ids):
    V, HA = aug.shape
    N = ids.shape[0]
    NC, NS = _sc_info()
    NW = NC * NS
    per_w = N // NW
    RC = 32
    chunks = per_w // RC

    mesh = plsc.VectorSubcoreMesh(core_axis_name="c", subcore_axis_name="s")

    @functools.partial(
        pl.kernel,
        mesh=mesh,
        out_type=jax.ShapeDtypeStruct((N, HA), jnp.float32),
        scratch_types=[
            pltpu.VMEM((RC,), jnp.int32),
            pltpu.VMEM((RC,), jnp.int32),
            pltpu.VMEM((RC, HA), jnp.float32),
            pltpu.VMEM((RC, HA), jnp.float32),
            pltpu.SemaphoreType.DMA,
            pltpu.SemaphoreType.DMA,
        ],
    )
    def sc_kernel(aug_hbm, ids_hbm, out_hbm,
                  idx_a, idx_b, rows_a, rows_b, gsem, wsem):
        wid = lax.axis_index("s") * NC + lax.axis_index("c")
        base = wid * per_w
        idx = [idx_a, idx_b]
        rows = [rows_a, rows_b]
        gh = [None] * chunks
        wh = [None] * chunks

        pltpu.sync_copy(ids_hbm.at[pl.ds(base, RC)], idx_a)
        gh[0] = pltpu.async_copy(aug_hbm.at[idx_a], rows_a, gsem)
        for c in range(chunks):
            cur = c % 2
            nxt = (c + 1) % 2
            gh[c].wait()
            if c >= 1:
                wh[c - 1].wait()
            if c + 1 < chunks:
                pltpu.sync_copy(
                    ids_hbm.at[pl.ds(base + (c + 1) * RC, RC)], idx[nxt])
                gh[c + 1] = pltpu.async_copy(
                    aug_hbm.at[idx[nxt]], rows[nxt], gsem)
            wh[c] = pltpu.async_copy(
                rows[cur], out_hbm.at[pl.ds(base + c * RC, RC)], wsem)
        wh[chunks - 1].wait()

    return sc_kernel(aug, ids)


def _cast_bf16(nw_aug, H):
    K, HA = nw_aug.shape
    NB = 8

    def body(in_ref, out_ref):
        out_ref[...] = in_ref[:, :H].astype(jnp.bfloat16)

    return pl.pallas_call(
        body,
        grid=(NB,),
        in_specs=[pl.BlockSpec((K // NB, HA), lambda i: (i, 0))],
        out_specs=pl.BlockSpec((K // NB, H), lambda i: (i, 0)),
        out_shape=jax.ShapeDtypeStruct((K, H), jnp.bfloat16),
    )(nw_aug)


def _tc_main(x, nw16, nb2, lpn2, w16, b_row, lp_row, tgt2):
    T, H = x.shape
    K, HA = nw16.shape
    V = w16.shape[0]
    TT = 128

    def body(x_ref, nw_ref, nb_ref, lpn_ref, w_ref, b_ref, lp_ref, tgt_ref,
             out4_ref, out1_ref, out3_ref, out2_ref, nw16_s):
        @pl.when(pl.program_id(0) == 0)
        def _cast_once():
            nw16_s[...] = nw_ref[:, :H].astype(jnp.bfloat16)

        xb16 = x_ref[...].astype(jnp.bfloat16)
        acc = lax.dot_general(
            xb16, nw16_s[...],
            (((1,), (1,)), ((), ())),
            preferred_element_type=jnp.float32)
        out4_ref[...] = acc + nb_ref[...]
        out1_ref[...] = jnp.broadcast_to(lpn_ref[...], (TT, K))
        logits = lax.dot_general(
            xb16, w_ref[...],
            (((1,), (1,)), ((), ())),
            preferred_element_type=jnp.float32)
        oh = lax.broadcasted_iota(jnp.int32, (TT, V), 1) == tgt_ref[...]
        zero = jnp.zeros((), jnp.float32)
        out3_ref[...] = (
            jnp.sum(jnp.where(oh, logits, zero), axis=1, keepdims=True)
            + jnp.sum(jnp.where(oh, jnp.broadcast_to(b_ref[...], (TT, V)),
                                zero), axis=1, keepdims=True))
        out2_ref[...] = jnp.sum(
            jnp.where(oh, jnp.broadcast_to(lp_ref[...], (TT, V)), zero),
            axis=1, keepdims=True)

    return pl.pallas_call(
        body,
        grid=(T // TT,),
        in_specs=[
            pl.BlockSpec((TT, H), lambda i: (i, 0)),
            pl.BlockSpec((K, HA), lambda i: (0, 0)),
            pl.BlockSpec((1, K), lambda i: (0, 0)),
            pl.BlockSpec((1, K), lambda i: (0, 0)),
            pl.BlockSpec((V, H), lambda i: (0, 0)),
            pl.BlockSpec((1, V), lambda i: (0, 0)),
            pl.BlockSpec((1, V), lambda i: (0, 0)),
            pl.BlockSpec((TT, 1), lambda i: (i, 0)),
        ],
        out_specs=[
            pl.BlockSpec((TT, K), lambda i: (i, 0)),
            pl.BlockSpec((TT, K), lambda i: (i, 0)),
            pl.BlockSpec((TT, 1), lambda i: (i, 0)),
            pl.BlockSpec((TT, 1), lambda i: (i, 0)),
        ],
        out_shape=[
            jax.ShapeDtypeStruct((T, K), jnp.float32),
            jax.ShapeDtypeStruct((T, K), jnp.float32),
            jax.ShapeDtypeStruct((T, 1), jnp.float32),
            jax.ShapeDtypeStruct((T, 1), jnp.float32),
        ],
        scratch_shapes=[pltpu.VMEM((K, H), jnp.bfloat16)],
    )(x, nw16, nb2, lpn2, w16, b_row, lp_row, tgt2)


def kernel(hidden, target, noise_samples, weight, bias, logprob_noise):
    seq_len, bsz, H = hidden.shape
    T = seq_len * bsz
    K = noise_samples.shape[0]
    V = weight.shape[0]
    x = hidden.reshape(T, H)
    tgt = target.reshape(T)

    pad = 128 - 2
    aug = jnp.concatenate(
        [weight, bias[:, None], logprob_noise[:, None],
         jnp.zeros((V, pad), jnp.float32)], axis=1)

    nw_aug = _sc_gather(aug, noise_samples)
    w16 = _cast_bf16(weight, H)
    nb2 = nw_aug[:, H].reshape(1, K)
    lpn2 = nw_aug[:, H + 1].reshape(1, K)
    out4, out1, out3, out2 = _tc_main(
        x, nw_aug, nb2, lpn2, w16, bias.reshape(1, V),
        logprob_noise.reshape(1, V), tgt.reshape(T, 1))
    return (out1, out2, out3, out4)


# speedup vs baseline: 1.6119x; 1.6119x over previous
import functools

import jax
import jax.numpy as jnp
from jax import lax
from jax.experimental import pallas as pl
from jax.experimental.pallas import tpu as pltpu, tpu_sc as plsc


def _sc_info():
    info = plsc.get_sparse_core_info()
    return info.num_cores, info.num_subcores


def _sc_gather(aug, ids, name_rows_per_chunk=32):
    V, HA = aug.shape
    N = ids.shape[0]
    NC, NS = _sc_info()
    NW = NC * NS
    per_w = N // NW
    RC = name_rows_per_chunk
    chunks = per_w // RC

    mesh = plsc.VectorSubcoreMesh(core_axis_name="c", subcore_axis_name="s")

    @functools.partial(
        pl.kernel,
        mesh=mesh,
        out_type=jax.ShapeDtypeStruct((N, HA), jnp.float32),
        scratch_types=[
            pltpu.VMEM((RC,), jnp.int32),
            pltpu.VMEM((RC,), jnp.int32),
            pltpu.VMEM((RC, HA), jnp.float32),
            pltpu.VMEM((RC, HA), jnp.float32),
            pltpu.SemaphoreType.DMA,
            pltpu.SemaphoreType.DMA,
        ],
    )
    def sc_kernel(aug_hbm, ids_hbm, out_hbm,
                  idx_a, idx_b, rows_a, rows_b, gsem, wsem):
        wid = lax.axis_index("s") * NC + lax.axis_index("c")
        base = wid * per_w
        idx = [idx_a, idx_b]
        rows = [rows_a, rows_b]
        gh = [None] * chunks
        wh = [None] * chunks

        pltpu.sync_copy(ids_hbm.at[pl.ds(base, RC)], idx_a)
        gh[0] = pltpu.async_copy(aug_hbm.at[idx_a], rows_a, gsem)
        for c in range(chunks):
            cur = c % 2
            nxt = (c + 1) % 2
            gh[c].wait()
            if c >= 1:
                wh[c - 1].wait()
            if c + 1 < chunks:
                pltpu.sync_copy(
                    ids_hbm.at[pl.ds(base + (c + 1) * RC, RC)], idx[nxt])
                gh[c + 1] = pltpu.async_copy(
                    aug_hbm.at[idx[nxt]], rows[nxt], gsem)
            wh[c] = pltpu.async_copy(
                rows[cur], out_hbm.at[pl.ds(base + c * RC, RC)], wsem)
        wh[chunks - 1].wait()

    return sc_kernel(aug, ids)


def _cast_bf16(nw_aug, H):
    K, HA = nw_aug.shape
    NB = 8

    def body(in_ref, out_ref):
        out_ref[...] = in_ref[:, :H].astype(jnp.bfloat16)

    return pl.pallas_call(
        body,
        grid=(NB,),
        in_specs=[pl.BlockSpec((K // NB, HA), lambda i: (i, 0))],
        out_specs=pl.BlockSpec((K // NB, H), lambda i: (i, 0)),
        out_shape=jax.ShapeDtypeStruct((K, H), jnp.bfloat16),
    )(nw_aug)


def _tc_main(x, nw16, nb2, lpn2, w16, b_row, lp_row, tgt2):
    T, H = x.shape
    K = nw16.shape[0]
    V = w16.shape[0]
    TT = 512

    def body(x_ref, nw_ref, nb_ref, lpn_ref, w_ref, b_ref, lp_ref, tgt_ref,
             out4_ref, out1_ref, out3_ref, out2_ref):
        xb16 = x_ref[...].astype(jnp.bfloat16)
        acc = lax.dot_general(
            xb16, nw_ref[...],
            (((1,), (1,)), ((), ())),
            preferred_element_type=jnp.float32)
        out4_ref[...] = acc + nb_ref[...]
        out1_ref[...] = jnp.broadcast_to(lpn_ref[...], (TT, K))
        logits = lax.dot_general(
            xb16, w_ref[...],
            (((1,), (1,)), ((), ())),
            preferred_element_type=jnp.float32)
        oh = lax.broadcasted_iota(jnp.int32, (TT, V), 1) == tgt_ref[...]
        zero = jnp.zeros((), jnp.float32)
        out3_ref[...] = (
            jnp.sum(jnp.where(oh, logits, zero), axis=1, keepdims=True)
            + jnp.sum(jnp.where(oh, jnp.broadcast_to(b_ref[...], (TT, V)),
                                zero), axis=1, keepdims=True))
        out2_ref[...] = jnp.sum(
            jnp.where(oh, jnp.broadcast_to(lp_ref[...], (TT, V)), zero),
            axis=1, keepdims=True)

    return pl.pallas_call(
        body,
        grid=(T // TT,),
        in_specs=[
            pl.BlockSpec((TT, H), lambda i: (i, 0)),
            pl.BlockSpec((K, H), lambda i: (0, 0)),
            pl.BlockSpec((1, K), lambda i: (0, 0)),
            pl.BlockSpec((1, K), lambda i: (0, 0)),
            pl.BlockSpec((V, H), lambda i: (0, 0)),
            pl.BlockSpec((1, V), lambda i: (0, 0)),
            pl.BlockSpec((1, V), lambda i: (0, 0)),
            pl.BlockSpec((TT, 1), lambda i: (i, 0)),
        ],
        out_specs=[
            pl.BlockSpec((TT, K), lambda i: (i, 0)),
            pl.BlockSpec((TT, K), lambda i: (i, 0)),
            pl.BlockSpec((TT, 1), lambda i: (i, 0)),
            pl.BlockSpec((TT, 1), lambda i: (i, 0)),
        ],
        out_shape=[
            jax.ShapeDtypeStruct((T, K), jnp.float32),
            jax.ShapeDtypeStruct((T, K), jnp.float32),
            jax.ShapeDtypeStruct((T, 1), jnp.float32),
            jax.ShapeDtypeStruct((T, 1), jnp.float32),
        ],
    )(x, nw16, nb2, lpn2, w16, b_row, lp_row, tgt2)


def kernel(hidden, target, noise_samples, weight, bias, logprob_noise):
    seq_len, bsz, H = hidden.shape
    T = seq_len * bsz
    K = noise_samples.shape[0]
    V = weight.shape[0]
    x = hidden.reshape(T, H)
    tgt = target.reshape(T)

    pad = 128 - 2
    aug = jnp.concatenate(
        [weight, bias[:, None], logprob_noise[:, None],
         jnp.zeros((V, pad), jnp.float32)], axis=1)

    nw_aug = _sc_gather(aug, noise_samples)
    nw16 = _cast_bf16(nw_aug, H)
    w16 = _cast_bf16(weight, H)
    nb2 = nw_aug[:, H].reshape(1, K)
    lpn2 = nw_aug[:, H + 1].reshape(1, K)
    out4, out1, out3, out2 = _tc_main(
        x, nw16, nb2, lpn2, w16, bias.reshape(1, V),
        logprob_noise.reshape(1, V), tgt.reshape(T, 1))
    return (out1, out2, out3, out4)


# speedup vs baseline: 1.6508x; 1.0242x over previous
import functools

import jax
import jax.numpy as jnp
from jax import lax
from jax.experimental import pallas as pl
from jax.experimental.pallas import tpu as pltpu, tpu_sc as plsc


def _sc_info():
    info = plsc.get_sparse_core_info()
    return info.num_cores, info.num_subcores


def _sc_gather(aug, ids):
    V, HA = aug.shape
    N = ids.shape[0]
    NC, NS = _sc_info()
    NW = NC * NS
    per_w = N // NW
    RC = 32
    chunks = per_w // RC

    mesh = plsc.VectorSubcoreMesh(core_axis_name="c", subcore_axis_name="s")

    @functools.partial(
        pl.kernel,
        mesh=mesh,
        out_type=jax.ShapeDtypeStruct((N, HA), jnp.float32),
        scratch_types=[
            pltpu.VMEM((RC,), jnp.int32),
            pltpu.VMEM((RC,), jnp.int32),
            pltpu.VMEM((RC, HA), jnp.float32),
            pltpu.VMEM((RC, HA), jnp.float32),
            pltpu.SemaphoreType.DMA,
            pltpu.SemaphoreType.DMA,
        ],
    )
    def sc_kernel(aug_hbm, ids_hbm, out_hbm,
                  idx_a, idx_b, rows_a, rows_b, gsem, wsem):
        wid = lax.axis_index("s") * NC + lax.axis_index("c")
        base = wid * per_w
        idx = [idx_a, idx_b]
        rows = [rows_a, rows_b]
        gh = [None] * chunks
        wh = [None] * chunks

        pltpu.sync_copy(ids_hbm.at[pl.ds(base, RC)], idx_a)
        gh[0] = pltpu.async_copy(aug_hbm.at[idx_a], rows_a, gsem)
        for c in range(chunks):
            cur = c % 2
            nxt = (c + 1) % 2
            gh[c].wait()
            if c >= 1:
                wh[c - 1].wait()
            if c + 1 < chunks:
                pltpu.sync_copy(
                    ids_hbm.at[pl.ds(base + (c + 1) * RC, RC)], idx[nxt])
                gh[c + 1] = pltpu.async_copy(
                    aug_hbm.at[idx[nxt]], rows[nxt], gsem)
            wh[c] = pltpu.async_copy(
                rows[cur], out_hbm.at[pl.ds(base + c * RC, RC)], wsem)
        wh[chunks - 1].wait()

    return sc_kernel(aug, ids)


def _cast_bf16(nw_aug, H):
    K, HA = nw_aug.shape
    NB = 8

    def body(in_ref, out_ref):
        out_ref[...] = in_ref[:, :H].astype(jnp.bfloat16)

    return pl.pallas_call(
        body,
        grid=(NB,),
        in_specs=[pl.BlockSpec((K // NB, HA), lambda i: (i, 0))],
        out_specs=pl.BlockSpec((K // NB, H), lambda i: (i, 0)),
        out_shape=jax.ShapeDtypeStruct((K, H), jnp.bfloat16),
    )(nw_aug)


def _tc_main(x, nw16, nb2, lpn2, w16, b_row, lp_row, tgt2):
    T, H = x.shape
    K, HA = nw16.shape
    V = w16.shape[0]
    TT = 256

    def body(x_ref, nw_ref, nb_ref, lpn_ref, w_ref, b_ref, lp_ref, tgt_ref,
             out4_ref, out1_ref, out3_ref, out2_ref, nw16_s):
        @pl.when(pl.program_id(0) == 0)
        def _cast_once():
            nw16_s[...] = nw_ref[:, :H].astype(jnp.bfloat16)

        xb16 = x_ref[...].astype(jnp.bfloat16)
        acc = lax.dot_general(
            xb16, nw16_s[...],
            (((1,), (1,)), ((), ())),
            preferred_element_type=jnp.float32)
        out4_ref[...] = acc + nb_ref[...]
        out1_ref[...] = jnp.broadcast_to(lpn_ref[...], (TT, K))
        logits = lax.dot_general(
            xb16, w_ref[...],
            (((1,), (1,)), ((), ())),
            preferred_element_type=jnp.float32)
        oh = lax.broadcasted_iota(jnp.int32, (TT, V), 1) == tgt_ref[...]
        zero = jnp.zeros((), jnp.float32)
        out3_ref[...] = (
            jnp.sum(jnp.where(oh, logits, zero), axis=1, keepdims=True)
            + jnp.sum(jnp.where(oh, jnp.broadcast_to(b_ref[...], (TT, V)),
                                zero), axis=1, keepdims=True))
        out2_ref[...] = jnp.sum(
            jnp.where(oh, jnp.broadcast_to(lp_ref[...], (TT, V)), zero),
            axis=1, keepdims=True)

    return pl.pallas_call(
        body,
        grid=(T // TT,),
        in_specs=[
            pl.BlockSpec((TT, H), lambda i: (i, 0)),
            pl.BlockSpec((K, HA), lambda i: (0, 0)),
            pl.BlockSpec((1, K), lambda i: (0, 0)),
            pl.BlockSpec((1, K), lambda i: (0, 0)),
            pl.BlockSpec((V, H), lambda i: (0, 0)),
            pl.BlockSpec((1, V), lambda i: (0, 0)),
            pl.BlockSpec((1, V), lambda i: (0, 0)),
            pl.BlockSpec((TT, 1), lambda i: (i, 0)),
        ],
        out_specs=[
            pl.BlockSpec((TT, K), lambda i: (i, 0)),
            pl.BlockSpec((TT, K), lambda i: (i, 0)),
            pl.BlockSpec((TT, 1), lambda i: (i, 0)),
            pl.BlockSpec((TT, 1), lambda i: (i, 0)),
        ],
        out_shape=[
            jax.ShapeDtypeStruct((T, K), jnp.float32),
            jax.ShapeDtypeStruct((T, K), jnp.float32),
            jax.ShapeDtypeStruct((T, 1), jnp.float32),
            jax.ShapeDtypeStruct((T, 1), jnp.float32),
        ],
        scratch_shapes=[pltpu.VMEM((K, H), jnp.bfloat16)],
    )(x, nw16, nb2, lpn2, w16, b_row, lp_row, tgt2)


def kernel(hidden, target, noise_samples, weight, bias, logprob_noise):
    seq_len, bsz, H = hidden.shape
    T = seq_len * bsz
    K = noise_samples.shape[0]
    V = weight.shape[0]
    x = hidden.reshape(T, H)
    tgt = target.reshape(T)

    pad = 128 - 2
    aug = jnp.concatenate(
        [weight, bias[:, None], logprob_noise[:, None],
         jnp.zeros((V, pad), jnp.float32)], axis=1)

    nw_aug = _sc_gather(aug, noise_samples)
    w16 = _cast_bf16(weight, H)
    nb2 = nw_aug[:, H].reshape(1, K)
    lpn2 = nw_aug[:, H + 1].reshape(1, K)
    out4, out1, out3, out2 = _tc_main(
        x, nw_aug, nb2, lpn2, w16, bias.reshape(1, V),
        logprob_noise.reshape(1, V), tgt.reshape(T, 1))
    return (out1, out2, out3, out4)


# speedup vs baseline: 1.6859x; 1.0213x over previous
import functools

import jax
import jax.numpy as jnp
from jax import lax
from jax.experimental import pallas as pl
from jax.experimental.pallas import tpu as pltpu, tpu_sc as plsc


def _sc_info():
    info = plsc.get_sparse_core_info()
    return info.num_cores, info.num_subcores


def _sc_gather(aug, ids):
    V, HA = aug.shape
    N = ids.shape[0]
    NC, NS = _sc_info()
    NW = NC * NS
    per_w = N // NW
    RC = 32
    chunks = per_w // RC

    mesh = plsc.VectorSubcoreMesh(core_axis_name="c", subcore_axis_name="s")

    @functools.partial(
        pl.kernel,
        mesh=mesh,
        out_type=jax.ShapeDtypeStruct((N, HA), jnp.float32),
        scratch_types=[
            pltpu.VMEM((RC,), jnp.int32),
            pltpu.VMEM((RC,), jnp.int32),
            pltpu.VMEM((RC, HA), jnp.float32),
            pltpu.VMEM((RC, HA), jnp.float32),
            pltpu.SemaphoreType.DMA,
            pltpu.SemaphoreType.DMA,
        ],
    )
    def sc_kernel(aug_hbm, ids_hbm, out_hbm,
                  idx_a, idx_b, rows_a, rows_b, gsem, wsem):
        wid = lax.axis_index("s") * NC + lax.axis_index("c")
        base = wid * per_w
        idx = [idx_a, idx_b]
        rows = [rows_a, rows_b]
        gh = [None] * chunks
        wh = [None] * chunks

        pltpu.sync_copy(ids_hbm.at[pl.ds(base, RC)], idx_a)
        gh[0] = pltpu.async_copy(aug_hbm.at[idx_a], rows_a, gsem)
        for c in range(chunks):
            cur = c % 2
            nxt = (c + 1) % 2
            gh[c].wait()
            if c >= 1:
                wh[c - 1].wait()
            if c + 1 < chunks:
                pltpu.sync_copy(
                    ids_hbm.at[pl.ds(base + (c + 1) * RC, RC)], idx[nxt])
                gh[c + 1] = pltpu.async_copy(
                    aug_hbm.at[idx[nxt]], rows[nxt], gsem)
            wh[c] = pltpu.async_copy(
                rows[cur], out_hbm.at[pl.ds(base + c * RC, RC)], wsem)
        wh[chunks - 1].wait()

    return sc_kernel(aug, ids)


def _tc_main(x, nw_aug, nb2, lpn2, w_f32, b_row, lp_row, tgt2):
    T, H = x.shape
    K, HA = nw_aug.shape
    V = w_f32.shape[0]
    TT = 256

    def body(x_ref, nw_ref, nb_ref, lpn_ref, w_ref, b_ref, lp_ref, tgt_ref,
             out4_ref, out1_ref, out3_ref, out2_ref, nw16_s, w16_s):
        @pl.when(pl.program_id(0) == 0)
        def _cast_once():
            nw16_s[...] = nw_ref[:, :H].astype(jnp.bfloat16)
            w16_s[...] = w_ref[...].astype(jnp.bfloat16)

        xb16 = x_ref[...].astype(jnp.bfloat16)
        acc = lax.dot_general(
            xb16, nw16_s[...],
            (((1,), (1,)), ((), ())),
            preferred_element_type=jnp.float32)
        out4_ref[...] = acc + nb_ref[...]
        out1_ref[...] = jnp.broadcast_to(lpn_ref[...], (TT, K))
        logits = lax.dot_general(
            xb16, w16_s[...],
            (((1,), (1,)), ((), ())),
            preferred_element_type=jnp.float32)
        oh = lax.broadcasted_iota(jnp.int32, (TT, V), 1) == tgt_ref[...]
        zero = jnp.zeros((), jnp.float32)
        out3_ref[...] = (
            jnp.sum(jnp.where(oh, logits, zero), axis=1, keepdims=True)
            + jnp.sum(jnp.where(oh, jnp.broadcast_to(b_ref[...], (TT, V)),
                                zero), axis=1, keepdims=True))
        out2_ref[...] = jnp.sum(
            jnp.where(oh, jnp.broadcast_to(lp_ref[...], (TT, V)), zero),
            axis=1, keepdims=True)

    return pl.pallas_call(
        body,
        grid=(T // TT,),
        in_specs=[
            pl.BlockSpec((TT, H), lambda i: (i, 0)),
            pl.BlockSpec((K, HA), lambda i: (0, 0)),
            pl.BlockSpec((1, K), lambda i: (0, 0)),
            pl.BlockSpec((1, K), lambda i: (0, 0)),
            pl.BlockSpec((V, H), lambda i: (0, 0)),
            pl.BlockSpec((1, V), lambda i: (0, 0)),
            pl.BlockSpec((1, V), lambda i: (0, 0)),
            pl.BlockSpec((TT, 1), lambda i: (i, 0)),
        ],
        out_specs=[
            pl.BlockSpec((TT, K), lambda i: (i, 0)),
            pl.BlockSpec((TT, K), lambda i: (i, 0)),
            pl.BlockSpec((TT, 1), lambda i: (i, 0)),
            pl.BlockSpec((TT, 1), lambda i: (i, 0)),
        ],
        out_shape=[
            jax.ShapeDtypeStruct((T, K), jnp.float32),
            jax.ShapeDtypeStruct((T, K), jnp.float32),
            jax.ShapeDtypeStruct((T, 1), jnp.float32),
            jax.ShapeDtypeStruct((T, 1), jnp.float32),
        ],
        scratch_shapes=[pltpu.VMEM((K, H), jnp.bfloat16),
                        pltpu.VMEM((V, H), jnp.bfloat16)],
    )(x, nw_aug, nb2, lpn2, w_f32, b_row, lp_row, tgt2)


def kernel(hidden, target, noise_samples, weight, bias, logprob_noise):
    seq_len, bsz, H = hidden.shape
    T = seq_len * bsz
    K = noise_samples.shape[0]
    V = weight.shape[0]
    x = hidden.reshape(T, H)
    tgt = target.reshape(T)

    pad = 128 - 2
    aug = jnp.concatenate(
        [weight, bias[:, None], logprob_noise[:, None],
         jnp.zeros((V, pad), jnp.float32)], axis=1)

    nw_aug = _sc_gather(aug, noise_samples)
    nb2 = nw_aug[:, H].reshape(1, K)
    lpn2 = nw_aug[:, H + 1].reshape(1, K)
    out4, out1, out3, out2 = _tc_main(
        x, nw_aug, nb2, lpn2, weight, bias.reshape(1, V),
        logprob_noise.reshape(1, V), tgt.reshape(T, 1))
    return (out1, out2, out3, out4)
